# baseline (speedup 1.0000x reference)
import jax
import jax.numpy as jnp
from jax.experimental import pallas as pl
from jax.experimental.pallas import tpu as pltpu

S_BLK = 512
NSLOT = 8


def _make_proj_kernel(ns, s_blk):
    look = NSLOT - 1

    def _proj_kernel(tok_ref, feat_hbm, w_ref, b_ref, out_hbm,
                     xbuf, ybuf, sem_in, sem_out):
        i = pl.program_id(0)
        n = pl.num_programs(0)
        slot = jax.lax.rem(i, NSLOT)

        def in_copy(step, slot_):
            j = jax.lax.div(step, ns)
            s = jax.lax.rem(step, ns)
            return pltpu.make_async_copy(
                feat_hbm.at[pl.ds(s * s_blk, s_blk), j, :],
                xbuf.at[slot_],
                sem_in.at[slot_],
            )

        def out_copy(step, slot_):
            j = jax.lax.div(step, ns)
            s = jax.lax.rem(step, ns)
            return pltpu.make_async_copy(
                ybuf.at[slot_],
                out_hbm.at[pl.ds(s * s_blk, s_blk), j, :],
                sem_out.at[slot_],
            )

        @pl.when(i == 0)
        def _():
            for k in range(look):
                in_copy(i + k, k).start()

        @pl.when(i + look < n)
        def _():
            in_copy(i + look, jax.lax.rem(i + look, NSLOT)).start()

        in_copy(i, slot).wait()

        @pl.when(i >= NSLOT)
        def _():
            out_copy(i - NSLOT, slot).wait()

        tok_j = tok_ref[jax.lax.div(i, ns)]
        x = xbuf[slot].astype(jnp.bfloat16)
        w = w_ref[tok_j]
        acc = jax.lax.dot_general(
            x, w,
            dimension_numbers=(((1,), (1,)), ((), ())),
            preferred_element_type=jnp.float32,
        )
        ybuf[slot] = acc + b_ref[tok_j]

        out_copy(i, slot).start()

        @pl.when(i == n - 1)
        def _():
            for k in range(1, NSLOT):
                out_copy(i - k, jax.lax.rem(i - k + NSLOT, NSLOT)).wait()
            out_copy(i, slot).wait()

    return _proj_kernel


def kernel(feat, tgt_lang_toks, W, b):
    S, B, C = feat.shape
    E, E_dim, _ = W.shape
    toks = tgt_lang_toks.astype(jnp.int32)
    ns = S // S_BLK

    b3 = b.reshape(E, 1, E_dim)
    w_bf = W.astype(jnp.bfloat16)

    grid_spec = pltpu.PrefetchScalarGridSpec(
        num_scalar_prefetch=1,
        grid=(B * ns,),
        in_specs=[
            pl.BlockSpec(memory_space=pl.ANY),
            pl.BlockSpec((E, E_dim, C), lambda i, tok: (0, 0, 0)),
            pl.BlockSpec((E, 1, E_dim), lambda i, tok: (0, 0, 0)),
        ],
        out_specs=pl.BlockSpec(memory_space=pl.ANY),
        scratch_shapes=[
            pltpu.VMEM((NSLOT, S_BLK, C), jnp.float32),
            pltpu.VMEM((NSLOT, S_BLK, E_dim), jnp.float32),
            pltpu.SemaphoreType.DMA((NSLOT,)),
            pltpu.SemaphoreType.DMA((NSLOT,)),
        ],
    )

    return pl.pallas_call(
        _make_proj_kernel(ns, S_BLK),
        grid_spec=grid_spec,
        out_shape=jax.ShapeDtypeStruct((S, B, E_dim), feat.dtype),
    )(toks, feat, w_bf, b3)


# speedup vs baseline: 1.0833x; 1.0833x over previous
import jax
import jax.numpy as jnp
from jax.experimental import pallas as pl
from jax.experimental.pallas import tpu as pltpu

S_BLK = 1024
NSLOT = 4


def _make_proj_kernel(ns, s_blk):
    look = NSLOT - 1

    def _proj_kernel(tok_ref, feat_hbm, w_ref, b_ref, out_hbm,
                     xbuf, ybuf, sem_in, sem_out):
        i = pl.program_id(0)
        n = pl.num_programs(0)
        slot = jax.lax.rem(i, NSLOT)

        def in_copy(step, slot_):
            j = jax.lax.div(step, ns)
            s = jax.lax.rem(step, ns)
            return pltpu.make_async_copy(
                feat_hbm.at[pl.ds(s * s_blk, s_blk), j, :],
                xbuf.at[slot_],
                sem_in.at[slot_],
            )

        def out_copy(step, slot_):
            j = jax.lax.div(step, ns)
            s = jax.lax.rem(step, ns)
            return pltpu.make_async_copy(
                ybuf.at[slot_],
                out_hbm.at[pl.ds(s * s_blk, s_blk), j, :],
                sem_out.at[slot_],
            )

        @pl.when(i == 0)
        def _():
            for k in range(look):
                in_copy(i + k, k).start()

        @pl.when(i + look < n)
        def _():
            in_copy(i + look, jax.lax.rem(i + look, NSLOT)).start()

        in_copy(i, slot).wait()

        @pl.when(i >= NSLOT)
        def _():
            out_copy(i - NSLOT, slot).wait()

        tok_j = tok_ref[jax.lax.div(i, ns)]
        x = xbuf[slot].astype(jnp.bfloat16)
        w = w_ref[tok_j]
        acc = jax.lax.dot_general(
            x, w,
            dimension_numbers=(((1,), (1,)), ((), ())),
            preferred_element_type=jnp.float32,
        )
        ybuf[slot] = acc + b_ref[tok_j]

        out_copy(i, slot).start()

        @pl.when(i == n - 1)
        def _():
            for k in range(1, NSLOT):
                out_copy(i - k, jax.lax.rem(i - k + NSLOT, NSLOT)).wait()
            out_copy(i, slot).wait()

    return _proj_kernel


def kernel(feat, tgt_lang_toks, W, b):
    S, B, C = feat.shape
    E, E_dim, _ = W.shape
    toks = tgt_lang_toks.astype(jnp.int32)
    ns = S // S_BLK

    b3 = b.reshape(E, 1, E_dim)
    w_bf = W.astype(jnp.bfloat16)

    grid_spec = pltpu.PrefetchScalarGridSpec(
        num_scalar_prefetch=1,
        grid=(B * ns,),
        in_specs=[
            pl.BlockSpec(memory_space=pl.ANY),
            pl.BlockSpec((E, E_dim, C), lambda i, tok: (0, 0, 0)),
            pl.BlockSpec((E, 1, E_dim), lambda i, tok: (0, 0, 0)),
        ],
        out_specs=pl.BlockSpec(memory_space=pl.ANY),
        scratch_shapes=[
            pltpu.VMEM((NSLOT, S_BLK, C), jnp.float32),
            pltpu.VMEM((NSLOT, S_BLK, E_dim), jnp.float32),
            pltpu.SemaphoreType.DMA((NSLOT,)),
            pltpu.SemaphoreType.DMA((NSLOT,)),
        ],
    )

    return pl.pallas_call(
        _make_proj_kernel(ns, S_BLK),
        grid_spec=grid_spec,
        out_shape=jax.ShapeDtypeStruct((S, B, E_dim), feat.dtype),
    )(toks, feat, w_bf, b3)
